# baseline (speedup 1.0000x reference)
import functools

import jax
import jax.numpy as jnp
import numpy as np
from jax import lax
from jax.experimental import pallas as pl
from jax.experimental.pallas import tpu as pltpu
from jax.experimental.pallas import tpu_sc as plsc

_NUM_CLASSES = 3
_NUM_ANCHORS = 3
_G = 76
_S = _G * _G
_NCH = 7 + _NUM_CLASSES
_STRIDE = 8.0
_ANCHOR_W = (11.0, 23.0, 37.0)
_ANCHOR_H = (14.0, 27.0, 58.0)

_B = 64
_UNITS = _B * _NUM_ANCHORS
_NW = 32
_UPW = _UNITS // _NW
_SCHUNK = 304
_NCHUNK = _S // _SCHUNK
_GPC = _SCHUNK // 16


def _sigmoid(v):
    return 1.0 / (1.0 + jnp.exp(-v))


def _splat_i32(v):
    return jnp.full((16,), v, jnp.int32)


def _sc_body(x_hbm, out_hbm, in_v, out_v):
    cid = lax.axis_index("c")
    sid = lax.axis_index("s")
    wid = sid * 2 + cid
    iota16 = lax.iota(jnp.int32, 16)
    ch_vec = [_splat_i32(c) for c in range(_NCH)]

    def unit_body(k, carry0):
        u = wid * _UPW + k
        b = u // _NUM_ANCHORS
        a = u - b * _NUM_ANCHORS
        aw = jnp.where(
            a == 0, _ANCHOR_W[0], jnp.where(a == 1, _ANCHOR_W[1], _ANCHOR_W[2])
        ).astype(jnp.float32)
        ah = jnp.where(
            a == 0, _ANCHOR_H[0], jnp.where(a == 1, _ANCHOR_H[1], _ANCHOR_H[2])
        ).astype(jnp.float32)
        pltpu.sync_copy(x_hbm.at[b, pl.ds(a * _NCH, _NCH)], in_v)
        obase = a * _S

        def chunk_body(ch, carry, aw=aw, ah=ah, b=b, obase=obase):
            row4 = ch * 4
            row4f = row4.astype(jnp.float32) * _STRIDE
            col = iota16
            row_loc = jnp.zeros((16,), jnp.int32)

            for g in range(_GPC):
                row = row4 + row_loc
                s_loc = iota16 + (g * 16)
                gx8 = col.astype(jnp.float32) * _STRIDE
                gyf = row_loc.astype(jnp.float32)

                v0 = plsc.load_gather(in_v, [ch_vec[0], row, col])
                r0 = _sigmoid(v0) * _STRIDE + gx8
                plsc.store_scatter(out_v, [s_loc, ch_vec[0]], r0)

                v1 = plsc.load_gather(in_v, [ch_vec[1], row, col])
                r1 = (_sigmoid(v1) + gyf) * _STRIDE + row4f
                plsc.store_scatter(out_v, [s_loc, ch_vec[1]], r1)

                v2 = plsc.load_gather(in_v, [ch_vec[2], row, col])
                r2 = jnp.minimum(jnp.exp(v2), 1000.0) * aw
                plsc.store_scatter(out_v, [s_loc, ch_vec[2]], r2)

                v3 = plsc.load_gather(in_v, [ch_vec[3], row, col])
                r3 = jnp.minimum(jnp.exp(v3), 1000.0) * ah
                plsc.store_scatter(out_v, [s_loc, ch_vec[3]], r3)

                for c in (4, 5):
                    vc = plsc.load_gather(in_v, [ch_vec[c], row, col])
                    plsc.store_scatter(out_v, [s_loc, ch_vec[c]], vc)

                for c in (6, 7, 8, 9):
                    vc = plsc.load_gather(in_v, [ch_vec[c], row, col])
                    plsc.store_scatter(out_v, [s_loc, ch_vec[c]], _sigmoid(vc))

                coln = col + 16
                wrap = coln >= _G
                col = jnp.where(wrap, coln - _G, coln)
                row_loc = row_loc + wrap.astype(jnp.int32)

            pltpu.sync_copy(
                out_v, out_hbm.at[b, pl.ds(obase + ch * _SCHUNK, _SCHUNK)]
            )
            return carry

        lax.fori_loop(0, _NCHUNK, chunk_body, 0)
        return carry0

    lax.fori_loop(0, _UPW, unit_body, 0)


@jax.jit
def kernel(x):
    B = x.shape[0]
    run = pl.kernel(
        _sc_body,
        mesh=plsc.VectorSubcoreMesh(core_axis_name="c", subcore_axis_name="s"),
        out_type=jax.ShapeDtypeStruct((B, _NUM_ANCHORS * _S, _NCH), jnp.float32),
        scratch_types=[
            pltpu.VMEM((_NCH, _G, _G), jnp.float32),
            pltpu.VMEM((_SCHUNK, _NCH), jnp.float32),
        ],
        compiler_params=pltpu.CompilerParams(
            needs_layout_passes=False, use_tc_tiling_on_sc=False
        ),
    )
    return run(x)


# speedup vs baseline: 1.8853x; 1.8853x over previous
import functools

import jax
import jax.numpy as jnp
from jax import lax
from jax.experimental import pallas as pl

_NUM_CLASSES = 3
_NUM_ANCHORS = 3
_IMG_SIZE = 608
_G = 76
_S = _G * _G
_NCH = 7 + _NUM_CLASSES
_STRIDE = _IMG_SIZE / _G
_ANCHOR_W = (11.0, 23.0, 37.0)
_ANCHOR_H = (14.0, 27.0, 58.0)


def _decode_kernel(x_ref, o_ref):
    a = pl.program_id(1)
    p = x_ref[0, 0]

    sig = jax.nn.sigmoid(p)
    expv = jnp.minimum(jnp.exp(p), 1000.0)

    c = lax.broadcasted_iota(jnp.int32, (_NCH, _S), 0)
    s = lax.broadcasted_iota(jnp.int32, (_NCH, _S), 1)
    gx = (s % _G).astype(jnp.float32)
    gy = (s // _G).astype(jnp.float32)

    aw = jnp.where(a == 0, _ANCHOR_W[0], jnp.where(a == 1, _ANCHOR_W[1], _ANCHOR_W[2]))
    ah = jnp.where(a == 0, _ANCHOR_H[0], jnp.where(a == 1, _ANCHOR_H[1], _ANCHOR_H[2]))
    aw = aw.astype(jnp.float32)
    ah = ah.astype(jnp.float32)

    val = jnp.where(
        c <= 1,
        (sig + jnp.where(c == 0, gx, gy)) * _STRIDE,
        jnp.where(
            c <= 3,
            expv * jnp.where(c == 2, aw, ah),
            jnp.where(c <= 5, p, sig),
        ),
    )
    o_ref[0, 0] = val.T


@jax.jit
def kernel(x):
    B = x.shape[0]
    xr = x.reshape(B, _NUM_ANCHORS, _NCH, _S)
    out = pl.pallas_call(
        _decode_kernel,
        grid=(B, _NUM_ANCHORS),
        in_specs=[
            pl.BlockSpec((1, 1, _NCH, _S), lambda b, a: (b, a, 0, 0)),
        ],
        out_specs=pl.BlockSpec((1, 1, _S, _NCH), lambda b, a: (b, a, 0, 0)),
        out_shape=jax.ShapeDtypeStruct((B, _NUM_ANCHORS, _S, _NCH), jnp.float32),
    )(xr)
    return out.reshape(B, _NUM_ANCHORS * _S, _NCH)
